# baseline (speedup 1.0000x reference)
import functools

import jax
import jax.numpy as jnp
from jax import lax
from jax.experimental import pallas as pl
from jax.experimental.pallas import tpu as pltpu
from jax.experimental.pallas import tpu_sc as plsc

N, C, H, W = 1000, 66, 28, 28
NCLS = C // 3
D = 3 * H * W
ROWS = N * NCLS
NC, NS, L = 2, 16, 16
NWORK = NC * NS
B_PAD = 1024
B_PER_W = B_PAD // NWORK

_mesh = plsc.VectorSubcoreMesh(core_axis_name="c", subcore_axis_name="s")


@functools.partial(
    pl.kernel,
    mesh=_mesh,
    out_type=jax.ShapeDtypeStruct((N, D), jnp.float32),
    scratch_types=[
        pltpu.VMEM((B_PER_W,), jnp.int32),
        pltpu.VMEM((B_PER_W, D), jnp.float32),
        pltpu.SemaphoreType.DMA,
    ],
    compiler_params=pltpu.CompilerParams(use_tc_tiling_on_sc=False),
)
def _gather_rows(table_hbm, labels_hbm, out_hbm, lbl_v, rows_v, sem):
    wid = lax.axis_index("s") * NC + lax.axis_index("c")
    base = wid * B_PER_W
    pltpu.sync_copy(labels_hbm.at[pl.ds(base, B_PER_W)], lbl_v)

    copies = []
    for i in range(B_PER_W // L):
        lbl = lbl_v[pl.ds(i * L, L)]
        n_idx = jnp.minimum(base + i * L + lax.iota(jnp.int32, L), N - 1)
        idx = n_idx * NCLS + lbl
        copies.append(
            pltpu.async_copy(table_hbm.at[idx], rows_v.at[pl.ds(i * L, L)], sem)
        )
    for cp in copies:
        cp.wait()

    @pl.when(wid < NWORK - 1)
    def _():
        pltpu.sync_copy(rows_v, out_hbm.at[pl.ds(base, B_PER_W)])

    @pl.when(wid == NWORK - 1)
    def _():
        tail = N - (NWORK - 1) * B_PER_W
        pltpu.sync_copy(rows_v.at[pl.ds(0, tail)], out_hbm.at[pl.ds(base, tail)])


def kernel(vert_pred, labels):
    table = vert_pred.reshape(ROWS, D)
    lbl = jnp.zeros((B_PAD,), jnp.int32).at[:N].set(labels.astype(jnp.int32))
    out = _gather_rows(table, lbl)
    return out.reshape(N, 3, H, W)


# speedup vs baseline: 1.8114x; 1.8114x over previous
import functools

import jax
import jax.numpy as jnp
from jax import lax
from jax.experimental import pallas as pl
from jax.experimental.pallas import tpu as pltpu
from jax.experimental.pallas import tpu_sc as plsc

N, C, H, W = 1000, 66, 28, 28
NCLS = C // 3
ROWS = N * C
NC, NS, L = 2, 16, 16
NWORK = NC * NS
B_PAD = 1024
B_PER_W = B_PAD // NWORK
K = 8

_mesh = plsc.VectorSubcoreMesh(core_axis_name="c", subcore_axis_name="s")


@functools.partial(
    pl.kernel,
    mesh=_mesh,
    out_type=jax.ShapeDtypeStruct((3 * N, H, W), jnp.float32),
    scratch_types=[
        pltpu.VMEM((B_PER_W,), jnp.int32),
        pltpu.VMEM((3 * K, H, W), jnp.float32),
        pltpu.SemaphoreType.DMA,
    ],
    compiler_params=pltpu.CompilerParams(
        use_tc_tiling_on_sc=True, needs_layout_passes=False
    ),
)
def _gather_blocks(table_hbm, labels_hbm, out_hbm, lbl_v, buf, sem):
    wid = lax.axis_index("s") * NC + lax.axis_index("c")
    base = wid * B_PER_W
    pltpu.sync_copy(labels_hbm.at[pl.ds(base, B_PER_W)], lbl_v)
    lanes = lax.iota(jnp.int32, L)

    for g in range(B_PER_W // L):
        lbl_vec = lbl_v[pl.ds(g * L, L)]
        n_vec = base + g * L + lanes
        row_vec = 3 * (n_vec * NCLS + lbl_vec)
        for cc in range(L // K):
            d0 = g * L + cc * K

            @pl.when(base + d0 < N)
            def _(d0=d0, cc=cc, row_vec=row_vec):
                copies = []
                for j in range(K):
                    i = cc * K + j
                    r0 = jnp.sum(jnp.where(lanes == i, row_vec, 0))
                    copies.append(
                        pltpu.async_copy(
                            table_hbm.at[pl.ds(r0, 3)],
                            buf.at[pl.ds(3 * j, 3)],
                            sem,
                        )
                    )
                for cp in copies:
                    cp.wait()
                pltpu.sync_copy(buf, out_hbm.at[pl.ds(3 * (base + d0), 3 * K)])


def kernel(vert_pred, labels):
    table = vert_pred.reshape(ROWS, H, W)
    lbl = jnp.zeros((B_PAD,), jnp.int32).at[:N].set(labels.astype(jnp.int32))
    out = _gather_blocks(table, lbl)
    return out.reshape(N, 3, H, W)


# speedup vs baseline: 11.9841x; 6.6160x over previous
import functools

import jax
import jax.numpy as jnp
from jax import lax
from jax.experimental import pallas as pl
from jax.experimental.pallas import tpu as pltpu
from jax.experimental.pallas import tpu_sc as plsc

N, C, H, W = 1000, 66, 28, 28
NCLS = C // 3
NC, NS, L = 2, 16, 16
NWORK = NC * NS
NPAD = 1024
NTASKS = H * W
BASE_T = NTASKS // NWORK
EXTRA = NTASKS % NWORK

_mesh = plsc.VectorSubcoreMesh(core_axis_name="c", subcore_axis_name="s")


@functools.partial(
    pl.kernel,
    mesh=_mesh,
    out_type=jax.ShapeDtypeStruct((3, H, W, NPAD), jnp.float32),
    scratch_types=[
        pltpu.VMEM((NPAD,), jnp.int32),
        pltpu.VMEM((C, N), jnp.float32),
        pltpu.VMEM((3 * NPAD,), jnp.float32),
        pltpu.SemaphoreType.DMA,
    ],
    compiler_params=pltpu.CompilerParams(
        use_tc_tiling_on_sc=True, needs_layout_passes=False
    ),
)
def _select_rows(b_hbm, labels_hbm, out_hbm, lbl_v, slab, obuf, sem):
    wid = lax.axis_index("s") * NC + lax.axis_index("c")
    pltpu.sync_copy(labels_hbm, lbl_v)
    lanes = lax.iota(jnp.int32, L)
    start = wid * BASE_T + jnp.minimum(wid, EXTRA)
    ntask = BASE_T + jnp.where(wid < EXTRA, 1, 0)

    def task(i, carry):
        t = start + i
        h = t // W
        w = t % W
        pltpu.sync_copy(b_hbm.at[h, w], slab)
        for g in range(NPAD // L):
            lbl = lbl_v[pl.ds(g * L, L)]
            col = jnp.minimum(g * L + lanes, N - 1)
            for c in range(3):
                row = 3 * lbl + c
                val = plsc.load_gather(slab, [row, col])
                obuf[pl.ds(c * NPAD + g * L, L)] = val
        copies = [
            pltpu.async_copy(
                obuf.at[pl.ds(c * NPAD, NPAD)], out_hbm.at[c, h, w], sem
            )
            for c in range(3)
        ]
        for cp in copies:
            cp.wait()
        return carry

    lax.fori_loop(0, ntask, task, None)


def kernel(vert_pred, labels):
    b = jnp.transpose(vert_pred, (2, 3, 1, 0))
    lbl = jnp.zeros((NPAD,), jnp.int32).at[:N].set(labels.astype(jnp.int32))
    out = _select_rows(b, lbl)
    return jnp.transpose(out[:, :, :, :N], (3, 0, 1, 2))


# speedup vs baseline: 13.8712x; 1.1575x over previous
import functools

import jax
import jax.numpy as jnp
from jax import lax
from jax.experimental import pallas as pl
from jax.experimental.pallas import tpu as pltpu
from jax.experimental.pallas import tpu_sc as plsc

N, C, H, W = 1000, 66, 28, 28
NC, NS, L = 2, 16, 16
NWORK = NC * NS
NPAD = 1024
CH = 512
NTASKS = H * W * 2
T_PER_W = NTASKS // NWORK
NPAIR = (T_PER_W + 1) // 2

_mesh = plsc.VectorSubcoreMesh(core_axis_name="c", subcore_axis_name="s")


@functools.partial(
    pl.kernel,
    mesh=_mesh,
    out_type=jax.ShapeDtypeStruct((3, H, W, N), jnp.float32),
    scratch_types=[
        pltpu.VMEM((NPAD,), jnp.int32),
        pltpu.VMEM((C, CH), jnp.float32),
        pltpu.VMEM((C, CH), jnp.float32),
        pltpu.VMEM((3 * CH,), jnp.float32),
        pltpu.VMEM((3 * CH,), jnp.float32),
        pltpu.SemaphoreType.DMA,
        pltpu.SemaphoreType.DMA,
        pltpu.SemaphoreType.DMA,
        pltpu.SemaphoreType.DMA,
    ],
    compiler_params=pltpu.CompilerParams(
        use_tc_tiling_on_sc=True,
        needs_layout_passes=False,
        disable_bounds_checks=True,
    ),
)
def _select_rows(
    b_hbm, labels_hbm, out_hbm,
    lbl_v, slab_a, slab_b, obuf_a, obuf_b, lsem_a, lsem_b, osem_a, osem_b,
):
    wid = lax.axis_index("s") * NC + lax.axis_index("c")
    pltpu.sync_copy(labels_hbm, lbl_v)
    lanes = lax.iota(jnp.int32, L)
    t0 = wid * T_PER_W

    def coords(t):
        p = t // 2
        return p // W, p % W, (t % 2) * CH

    def issue_load(t, slab, lsem):
        h, w, o = coords(t)
        pltpu.async_copy(b_hbm.at[h, w, :, pl.ds(o, CH)], slab, lsem)

    def drain_load(slab, lsem):
        pltpu.make_async_copy(
            b_hbm.at[0, 0, :, pl.ds(0, CH)], slab, lsem
        ).wait()

    def gather(t, slab, obuf):
        h, w, o = coords(t)
        for g in range(CH // L):
            lbl = lbl_v[pl.ds(o + g * L, L)]
            col = g * L + lanes
            for c in range(3):
                row = 3 * lbl + c
                obuf[pl.ds(c * CH + g * L, L)] = plsc.load_gather(
                    slab, [row, col]
                )

    def issue_out(t, obuf, osem):
        h, w, o = coords(t)
        for c in range(3):
            pltpu.async_copy(
                obuf.at[pl.ds(c * CH, CH)],
                out_hbm.at[c, h, w, pl.ds(o, CH)],
                osem,
            )

    def drain_out(obuf, osem):
        for c in range(3):
            pltpu.make_async_copy(
                obuf.at[pl.ds(c * CH, CH)],
                out_hbm.at[c, 0, 0, pl.ds(0, CH)],
                osem,
            ).wait()

    issue_load(t0, slab_a, lsem_a)
    issue_load(t0 + 1, slab_b, lsem_b)

    def step(i, carry):
        ta = t0 + 2 * i

        @pl.when(i > 0)
        def _():
            drain_out(obuf_a, osem_a)

        drain_load(slab_a, lsem_a)
        gather(ta, slab_a, obuf_a)

        @pl.when(2 * i + 2 < T_PER_W)
        def _():
            issue_load(ta + 2, slab_a, lsem_a)

        issue_out(ta, obuf_a, osem_a)

        @pl.when(2 * i + 1 < T_PER_W)
        def _():
            tb = ta + 1

            @pl.when(i > 0)
            def _():
                drain_out(obuf_b, osem_b)

            drain_load(slab_b, lsem_b)
            gather(tb, slab_b, obuf_b)

            @pl.when(2 * i + 3 < T_PER_W)
            def _():
                issue_load(tb + 2, slab_b, lsem_b)

            issue_out(tb, obuf_b, osem_b)

        return carry

    lax.fori_loop(0, NPAIR, step, None)
    drain_out(obuf_a, osem_a)
    drain_out(obuf_b, osem_b)


def kernel(vert_pred, labels):
    b = jnp.transpose(vert_pred, (2, 3, 1, 0))
    lbl = jnp.zeros((NPAD,), jnp.int32).at[:N].set(labels.astype(jnp.int32))
    out = _select_rows(b, lbl)
    return jnp.transpose(out, (3, 0, 1, 2))


# speedup vs baseline: 13.9051x; 1.0024x over previous
import functools

import jax
import jax.numpy as jnp
from jax import lax
from jax.experimental import pallas as pl
from jax.experimental.pallas import tpu as pltpu
from jax.experimental.pallas import tpu_sc as plsc

N, C, H, W = 1000, 66, 28, 28
NC, NS, L = 2, 16, 16
NWORK = NC * NS
NPAD = 1024
CH = 512
NTASKS = H * W * 2
T_PER_W = NTASKS // NWORK
NPAIR = (T_PER_W + 1) // 2

_mesh = plsc.VectorSubcoreMesh(core_axis_name="c", subcore_axis_name="s")


@functools.partial(
    pl.kernel,
    mesh=_mesh,
    out_type=jax.ShapeDtypeStruct((3, H, W, N), jnp.float32),
    scratch_types=[
        pltpu.VMEM((NPAD,), jnp.int32),
        pltpu.VMEM((C, CH), jnp.float32),
        pltpu.VMEM((C, CH), jnp.float32),
        pltpu.VMEM((3, CH), jnp.float32),
        pltpu.VMEM((3, CH), jnp.float32),
        pltpu.SemaphoreType.DMA,
        pltpu.SemaphoreType.DMA,
        pltpu.SemaphoreType.DMA,
        pltpu.SemaphoreType.DMA,
    ],
    compiler_params=pltpu.CompilerParams(
        use_tc_tiling_on_sc=True,
        needs_layout_passes=False,
        disable_bounds_checks=True,
    ),
)
def _select_rows(
    b_hbm, labels_hbm, out_hbm,
    lbl_v, slab_a, slab_b, obuf_a, obuf_b, lsem_a, lsem_b, osem_a, osem_b,
):
    wid = lax.axis_index("s") * NC + lax.axis_index("c")
    pltpu.sync_copy(labels_hbm, lbl_v)
    lanes = lax.iota(jnp.int32, L)
    t0 = wid * T_PER_W

    def coords(t):
        p = t // 2
        return p // W, p % W, (t % 2) * CH

    def issue_load(t, slab, lsem):
        h, w, o = coords(t)
        pltpu.async_copy(b_hbm.at[h, w, :, pl.ds(o, CH)], slab, lsem)

    def drain_load(slab, lsem):
        pltpu.make_async_copy(
            b_hbm.at[0, 0, :, pl.ds(0, CH)], slab, lsem
        ).wait()

    def gather(t, slab, obuf):
        h, w, o = coords(t)
        for g in range(CH // L):
            base = 3 * lbl_v[pl.ds(o + g * L, L)]
            col = g * L + lanes
            for c in range(3):
                obuf[c, pl.ds(g * L, L)] = plsc.load_gather(
                    slab, [base + c, col]
                )

    def issue_out(t, obuf, osem):
        h, w, o = coords(t)
        pltpu.async_copy(obuf, out_hbm.at[:, h, w, pl.ds(o, CH)], osem)

    def drain_out(obuf, osem):
        pltpu.make_async_copy(
            obuf, out_hbm.at[:, 0, 0, pl.ds(0, CH)], osem
        ).wait()

    issue_load(t0, slab_a, lsem_a)
    issue_load(t0 + 1, slab_b, lsem_b)

    def step(i, carry):
        ta = t0 + 2 * i

        @pl.when(i > 0)
        def _():
            drain_out(obuf_a, osem_a)

        drain_load(slab_a, lsem_a)
        gather(ta, slab_a, obuf_a)

        @pl.when(2 * i + 2 < T_PER_W)
        def _():
            issue_load(ta + 2, slab_a, lsem_a)

        issue_out(ta, obuf_a, osem_a)

        @pl.when(2 * i + 1 < T_PER_W)
        def _():
            tb = ta + 1

            @pl.when(i > 0)
            def _():
                drain_out(obuf_b, osem_b)

            drain_load(slab_b, lsem_b)
            gather(tb, slab_b, obuf_b)

            @pl.when(2 * i + 3 < T_PER_W)
            def _():
                issue_load(tb + 2, slab_b, lsem_b)

            issue_out(tb, obuf_b, osem_b)

        return carry

    lax.fori_loop(0, NPAIR, step, None)
    drain_out(obuf_a, osem_a)
    drain_out(obuf_b, osem_b)


def kernel(vert_pred, labels):
    b = jnp.transpose(vert_pred, (2, 3, 1, 0))
    lbl = jnp.zeros((NPAD,), jnp.int32).at[:N].set(labels.astype(jnp.int32))
    out = _select_rows(b, lbl)
    return jnp.transpose(out, (3, 0, 1, 2))


# speedup vs baseline: 15.5305x; 1.1169x over previous
import functools

import jax
import jax.numpy as jnp
from jax import lax
from jax.experimental import pallas as pl
from jax.experimental.pallas import tpu as pltpu
from jax.experimental.pallas import tpu_sc as plsc

N, C, H, W = 1000, 66, 28, 28
NC, NS, L = 2, 16, 16
NWORK = NC * NS
NPAD = 1024
CH = 128
CPP = NPAD // CH
NBUF = 4
NTASKS = H * W * CPP
T_PER_W = NTASKS // NWORK
NITER = T_PER_W // NBUF

_mesh = plsc.VectorSubcoreMesh(core_axis_name="c", subcore_axis_name="s")


@functools.partial(
    pl.kernel,
    mesh=_mesh,
    out_type=jax.ShapeDtypeStruct((3, H, W, N), jnp.float32),
    scratch_types=(
        [pltpu.VMEM((NPAD,), jnp.int32)]
        + [pltpu.VMEM((C, CH), jnp.float32) for _ in range(NBUF)]
        + [pltpu.VMEM((3, CH), jnp.float32) for _ in range(NBUF)]
        + [pltpu.SemaphoreType.DMA for _ in range(2 * NBUF)]
    ),
    compiler_params=pltpu.CompilerParams(
        use_tc_tiling_on_sc=True,
        needs_layout_passes=False,
        disable_bounds_checks=True,
    ),
)
def _select_rows(b_hbm, labels_hbm, out_hbm, lbl_v, *bufs):
    slabs = bufs[:NBUF]
    obufs = bufs[NBUF : 2 * NBUF]
    lsems = bufs[2 * NBUF : 3 * NBUF]
    osems = bufs[3 * NBUF :]

    wid = lax.axis_index("s") * NC + lax.axis_index("c")
    pltpu.sync_copy(labels_hbm, lbl_v)
    lanes = lax.iota(jnp.int32, L)
    t0 = wid * T_PER_W

    def coords(t):
        p = t // CPP
        return p // W, p % W, (t % CPP) * CH

    def issue_load(t, b):
        h, w, o = coords(t)
        pltpu.async_copy(b_hbm.at[h, w, :, pl.ds(o, CH)], slabs[b], lsems[b])

    def drain_load(b):
        pltpu.make_async_copy(
            b_hbm.at[0, 0, :, pl.ds(0, CH)], slabs[b], lsems[b]
        ).wait()

    def gather(t, b):
        _, _, o = coords(t)
        for g in range(CH // L):
            base = 3 * lbl_v[pl.ds(o + g * L, L)]
            col = g * L + lanes
            for c in range(3):
                obufs[b][c, pl.ds(g * L, L)] = plsc.load_gather(
                    slabs[b], [base + c, col]
                )

    def issue_out(t, b):
        h, w, o = coords(t)
        pltpu.async_copy(obufs[b], out_hbm.at[:, h, w, pl.ds(o, CH)], osems[b])

    def drain_out(b):
        pltpu.make_async_copy(
            obufs[b], out_hbm.at[:, 0, 0, pl.ds(0, CH)], osems[b]
        ).wait()

    for b in range(NBUF):
        issue_load(t0 + b, b)

    def step(i, carry):
        for b in range(NBUF):
            t = t0 + i * NBUF + b

            @pl.when(i > 0)
            def _(b=b):
                drain_out(b)

            drain_load(b)
            gather(t, b)

            @pl.when(i < NITER - 1)
            def _(t=t, b=b):
                issue_load(t + NBUF, b)

            issue_out(t, b)
        return carry

    lax.fori_loop(0, NITER, step, None)
    for b in range(NBUF):
        drain_out(b)


def kernel(vert_pred, labels):
    b = jnp.transpose(vert_pred, (2, 3, 1, 0))
    lbl = jnp.zeros((NPAD,), jnp.int32).at[:N].set(labels.astype(jnp.int32))
    out = _select_rows(b, lbl)
    return jnp.transpose(out, (3, 0, 1, 2))


# speedup vs baseline: 17.1798x; 1.1062x over previous
import functools

import jax
import jax.numpy as jnp
from jax import lax
from jax.experimental import pallas as pl
from jax.experimental.pallas import tpu as pltpu
from jax.experimental.pallas import tpu_sc as plsc

N, C, H, W = 1000, 66, 28, 28
NC, NS, L = 2, 16, 16
NWORK = NC * NS
NPAD = 1024
CH = 128
CPP = NPAD // CH
NBUF = 7
NTASKS = H * W * CPP
T_PER_W = NTASKS // NWORK
NITER = T_PER_W // NBUF

_mesh = plsc.VectorSubcoreMesh(core_axis_name="c", subcore_axis_name="s")


@functools.partial(
    pl.kernel,
    mesh=_mesh,
    out_type=jax.ShapeDtypeStruct((3, H, W, N), jnp.float32),
    scratch_types=(
        [pltpu.VMEM((NPAD,), jnp.int32)]
        + [pltpu.VMEM((C, CH), jnp.float32) for _ in range(NBUF)]
        + [pltpu.VMEM((3, CH), jnp.float32) for _ in range(NBUF)]
        + [pltpu.SemaphoreType.DMA for _ in range(2 * NBUF)]
    ),
    compiler_params=pltpu.CompilerParams(
        use_tc_tiling_on_sc=True,
        needs_layout_passes=False,
        disable_bounds_checks=True,
    ),
)
def _select_rows(b_hbm, labels_hbm, out_hbm, lbl_v, *bufs):
    slabs = bufs[:NBUF]
    obufs = bufs[NBUF : 2 * NBUF]
    lsems = bufs[2 * NBUF : 3 * NBUF]
    osems = bufs[3 * NBUF :]

    wid = lax.axis_index("s") * NC + lax.axis_index("c")
    pltpu.sync_copy(labels_hbm, lbl_v)
    lanes = lax.iota(jnp.int32, L)
    t0 = wid * T_PER_W

    def coords(t):
        p = t // CPP
        return p // W, p % W, (t % CPP) * CH

    def issue_load(t, b):
        h, w, o = coords(t)
        pltpu.async_copy(b_hbm.at[h, w, :, pl.ds(o, CH)], slabs[b], lsems[b])

    def drain_load(b):
        pltpu.make_async_copy(
            b_hbm.at[0, 0, :, pl.ds(0, CH)], slabs[b], lsems[b]
        ).wait()

    def gather(t, b):
        _, _, o = coords(t)
        for g in range(CH // L):
            base = 3 * lbl_v[pl.ds(o + g * L, L)]
            col = g * L + lanes
            for c in range(3):
                obufs[b][c, pl.ds(g * L, L)] = plsc.load_gather(
                    slabs[b], [base + c, col]
                )

    def issue_out(t, b):
        h, w, o = coords(t)
        pltpu.async_copy(obufs[b], out_hbm.at[:, h, w, pl.ds(o, CH)], osems[b])

    def drain_out(b):
        pltpu.make_async_copy(
            obufs[b], out_hbm.at[:, 0, 0, pl.ds(0, CH)], osems[b]
        ).wait()

    for b in range(NBUF):
        issue_load(t0 + b, b)

    def step(i, carry):
        for b in range(NBUF):
            t = t0 + i * NBUF + b

            @pl.when(i > 0)
            def _(b=b):
                drain_out(b)

            drain_load(b)
            gather(t, b)

            @pl.when(i < NITER - 1)
            def _(t=t, b=b):
                issue_load(t + NBUF, b)

            issue_out(t, b)
        return carry

    lax.fori_loop(0, NITER, step, None)
    for b in range(NBUF):
        drain_out(b)


def kernel(vert_pred, labels):
    b = jnp.transpose(vert_pred, (2, 3, 1, 0))
    lbl = jnp.zeros((NPAD,), jnp.int32).at[:N].set(labels.astype(jnp.int32))
    out = _select_rows(b, lbl)
    return jnp.transpose(out, (3, 0, 1, 2))
